# baseline (device time: 89960 ns/iter reference)
import jax
import jax.numpy as jnp
from jax import lax
from jax.experimental import pallas as pl
from jax.experimental.pallas import tpu as pltpu

N_DEV = 8
SQ = 1024
D = 1024
HQ = 8
DH = 128
BLK = SQ // N_DEV
BAND = 384
WIN = 128
HALO = 128
SCALE = 0.08838834764831843
N_FWD = 4
N_BWD = 3
N_QREM = 6


def kernel(x, Wq, K_ext, V_ext, Wo):
    x2 = x.reshape(SQ, D)
    k2 = K_ext.reshape(K_ext.shape[1], HQ * DH)
    v2 = V_ext.reshape(V_ext.shape[1], HQ * DH)

    def body(x_ref, wq_ref, k_ref, v_ref, wo_ref, out_ref,
             ctx_scr, ctx_slice, khalo, vhalo, fstage, bstage,
             qsend, qstage,
             halo_send, halo_recv, scat_send, scat_recv,
             fsend, frecv, bsend, brecv, qsend_sem, qrecv):
        pos = lax.axis_index("i")
        right = (pos + 1) % N_DEV
        left = (pos - 1) % N_DEV

        @pl.when(pos == 1)
        def _():
            k_rdma = pltpu.make_async_remote_copy(
                src_ref=k_ref.at[pl.ds(0, HALO), :], dst_ref=khalo,
                send_sem=halo_send.at[0], recv_sem=halo_recv.at[0],
                device_id=(0,), device_id_type=pl.DeviceIdType.MESH)
            v_rdma = pltpu.make_async_remote_copy(
                src_ref=v_ref.at[pl.ds(0, HALO), :], dst_ref=vhalo,
                send_sem=halo_send.at[1], recv_sem=halo_recv.at[1],
                device_id=(0,), device_id_type=pl.DeviceIdType.MESH)
            k_rdma.start()
            v_rdma.start()
            k_rdma.wait_send()
            v_rdma.wait_send()

        for d in range(2, N_DEV):
            @pl.when(pos == d)
            def _(d=d):
                qsend[...] = jnp.dot(
                    x_ref[pl.ds(d * BLK, BLK), :], wq_ref[...],
                    preferred_element_type=jnp.float32)
                q_rdma = pltpu.make_async_remote_copy(
                    src_ref=qsend,
                    dst_ref=qstage.at[pl.ds((d - 2) * BLK, BLK), :],
                    send_sem=qsend_sem, recv_sem=qrecv.at[d - 2],
                    device_id=(0,), device_id_type=pl.DeviceIdType.MESH)
                q_rdma.start()
                q_rdma.wait_send()

        @pl.when(pos == 0)
        def _():
            k_wait = pltpu.make_async_remote_copy(
                src_ref=k_ref.at[pl.ds(0, HALO), :], dst_ref=khalo,
                send_sem=halo_send.at[0], recv_sem=halo_recv.at[0],
                device_id=(1,), device_id_type=pl.DeviceIdType.MESH)
            v_wait = pltpu.make_async_remote_copy(
                src_ref=v_ref.at[pl.ds(0, HALO), :], dst_ref=vhalo,
                send_sem=halo_send.at[1], recv_sem=halo_recv.at[1],
                device_id=(1,), device_id_type=pl.DeviceIdType.MESH)

            q01 = jnp.dot(x_ref[pl.ds(0, 2 * BLK), :], wq_ref[...],
                          preferred_element_type=jnp.float32)

            r_i = lax.broadcasted_iota(jnp.int32, (BLK, BAND), 0)
            c_i = lax.broadcasted_iota(jnp.int32, (BLK, BAND), 1)
            mask0 = jnp.abs(r_i - c_i) <= WIN
            maskn = jnp.abs(r_i - c_i + WIN) <= WIN

            rdmas = []
            for qb in range(N_DEV):
                s = max(0, BLK * qb - WIN)
                if qb < 2:
                    q_blk = q01[qb * BLK:(qb + 1) * BLK, :]
                else:
                    q_wait = pltpu.make_async_remote_copy(
                        src_ref=qsend,
                        dst_ref=qstage.at[pl.ds((qb - 2) * BLK, BLK), :],
                        send_sem=qsend_sem, recv_sem=qrecv.at[qb - 2],
                        device_id=(qb,), device_id_type=pl.DeviceIdType.MESH)
                    q_wait.wait_recv()
                    q_blk = qstage[pl.ds((qb - 2) * BLK, BLK), :]
                if qb < N_DEV - 1:
                    kband = k_ref[pl.ds(s, BAND), :]
                    vband = v_ref[pl.ds(s, BAND), :]
                else:
                    k_wait.wait_recv()
                    v_wait.wait_recv()
                    kband = jnp.concatenate(
                        [k_ref[pl.ds(s, BAND - HALO), :], khalo[...]], axis=0)
                    vband = jnp.concatenate(
                        [v_ref[pl.ds(s, BAND - HALO), :], vhalo[...]], axis=0)
                mask = mask0 if qb == 0 else maskn
                for h in range(HQ):
                    qh = q_blk[:, h * DH:(h + 1) * DH]
                    kb = kband[:, h * DH:(h + 1) * DH]
                    vb = vband[:, h * DH:(h + 1) * DH]
                    scores = lax.dot_general(
                        qh, kb, (((1,), (1,)), ((), ())),
                        preferred_element_type=jnp.float32) * SCALE
                    scores = jnp.where(mask, scores, -1e9)
                    m = jnp.max(scores, axis=1, keepdims=True)
                    w = jnp.exp(scores - m)
                    l = jnp.sum(w, axis=1, keepdims=True)
                    ctx_h = jnp.dot(w, vb, preferred_element_type=jnp.float32)
                    ctx_scr[pl.ds(qb * BLK, BLK), pl.ds(h * DH, DH)] = (
                        ctx_h / l)
                if qb != 0:
                    r = pltpu.make_async_remote_copy(
                        src_ref=ctx_scr.at[pl.ds(qb * BLK, BLK), :],
                        dst_ref=ctx_slice,
                        send_sem=scat_send.at[qb - 1], recv_sem=scat_recv,
                        device_id=(qb,), device_id_type=pl.DeviceIdType.MESH)
                    r.start()
                    rdmas.append(r)
            ctx_slice[...] = ctx_scr[pl.ds(0, BLK), :]
            for r in rdmas:
                r.wait_send()

        @pl.when(pos != 0)
        def _():
            scat_wait = pltpu.make_async_remote_copy(
                src_ref=ctx_slice, dst_ref=ctx_slice,
                send_sem=scat_send.at[0], recv_sem=scat_recv,
                device_id=(0,), device_id_type=pl.DeviceIdType.MESH)
            scat_wait.wait_recv()

        out_slice = jnp.dot(ctx_slice[...], wo_ref[...],
                            preferred_element_type=jnp.float32)
        out_ref[pl.ds(pos * BLK, BLK), :] = out_slice
        fstage[0] = out_slice
        bstage[0] = out_slice

        f_rdma = [
            pltpu.make_async_remote_copy(
                src_ref=fstage.at[h], dst_ref=fstage.at[h + 1],
                send_sem=fsend.at[h], recv_sem=frecv.at[h],
                device_id=(right,), device_id_type=pl.DeviceIdType.MESH)
            for h in range(N_FWD)]
        b_rdma = [
            pltpu.make_async_remote_copy(
                src_ref=bstage.at[h], dst_ref=bstage.at[h + 1],
                send_sem=bsend.at[h], recv_sem=brecv.at[h],
                device_id=(left,), device_id_type=pl.DeviceIdType.MESH)
            for h in range(N_BWD)]
        f_rdma[0].start()
        b_rdma[0].start()
        for h in range(N_FWD):
            f_rdma[h].wait_recv()
            if h + 1 < N_FWD:
                f_rdma[h + 1].start()
            if h < N_BWD:
                b_rdma[h].wait_recv()
                if h + 1 < N_BWD:
                    b_rdma[h + 1].start()
            out_ref[pl.ds(((pos - h - 1) % N_DEV) * BLK, BLK), :] = (
                fstage[h + 1])
            if h < N_BWD:
                out_ref[pl.ds(((pos + h + 1) % N_DEV) * BLK, BLK), :] = (
                    bstage[h + 1])
        for r in f_rdma:
            r.wait_send()
        for r in b_rdma:
            r.wait_send()

    out = pl.pallas_call(
        body,
        out_shape=jax.ShapeDtypeStruct((SQ, D), jnp.float32),
        in_specs=[pl.BlockSpec(memory_space=pltpu.VMEM)] * 5,
        out_specs=pl.BlockSpec(memory_space=pltpu.VMEM),
        scratch_shapes=[
            pltpu.VMEM((SQ, D), jnp.float32),
            pltpu.VMEM((BLK, D), jnp.float32),
            pltpu.VMEM((HALO, HQ * DH), jnp.float32),
            pltpu.VMEM((HALO, HQ * DH), jnp.float32),
            pltpu.VMEM((N_FWD + 1, BLK, D), jnp.float32),
            pltpu.VMEM((N_BWD + 1, BLK, D), jnp.float32),
            pltpu.VMEM((BLK, D), jnp.float32),
            pltpu.VMEM((N_QREM * BLK, D), jnp.float32),
            pltpu.SemaphoreType.DMA((2,)),
            pltpu.SemaphoreType.DMA((2,)),
            pltpu.SemaphoreType.DMA((N_DEV - 1,)),
            pltpu.SemaphoreType.DMA,
            pltpu.SemaphoreType.DMA((N_FWD,)),
            pltpu.SemaphoreType.DMA((N_FWD,)),
            pltpu.SemaphoreType.DMA((N_BWD,)),
            pltpu.SemaphoreType.DMA((N_BWD,)),
            pltpu.SemaphoreType.DMA,
            pltpu.SemaphoreType.DMA((N_QREM,)),
        ],
    )(x2, Wq, k2, v2, Wo)
    return out.reshape(1, SQ, D)
